# baseline (speedup 1.0000x reference)
import jax
import jax.numpy as jnp
from jax.experimental import pallas as pl
from jax.experimental.pallas import tpu as pltpu


def _update_body(pos_ref,
                 ck0, cv0, ck1, cv1, ck2, cv2, ck3, cv3,
                 ks0, vs0, ks1, vs1, ks2, vs2, ks3, vs3,
                 ok0, ov0, ok1, ov1, ok2, ov2, ok3, ov3,
                 vt0, vt1, vt2, vt3,
                 *sems):
    p = pos_ref[0]
    kc = ((ck0, ks0, ok0), (ck1, ks1, ok1), (ck2, ks2, ok2), (ck3, ks3, ok3))
    vc = ((cv0, vs0, ov0, vt0), (cv1, vs1, ov1, vt1),
          (cv2, vs2, ov2, vt2), (cv3, vs3, ov3, vt3))

    bulk = []
    for i, (ck, _, ok) in enumerate(kc):
        c = pltpu.make_async_copy(ck, ok, sems[i])
        c.start()
        bulk.append(c)
    for i, (cv, _, ov, _) in enumerate(vc):
        c = pltpu.make_async_copy(cv, ov, sems[4 + i])
        c.start()
        bulk.append(c)

    aligned = pl.multiple_of((p // 128) * 128, 128)
    col = p - aligned
    fetches = []
    for i, (cv, _, _, vt) in enumerate(vc):
        c = pltpu.make_async_copy(cv.at[:, :, :, pl.ds(aligned, 128)], vt, sems[8 + i])
        c.start()
        fetches.append(c)
    lane = jax.lax.broadcasted_iota(jnp.int32, vt0.shape, 3)
    for i, (_, vs, _, vt) in enumerate(vc):
        fetches[i].wait()
        vt[...] = jnp.where(lane == col, vs[...], vt[...])

    tails = []
    for i, (_, ks, ok) in enumerate(kc):
        bulk[i].wait()
        c = pltpu.make_async_copy(ks, ok.at[:, :, pl.ds(p, 1), :], sems[i])
        c.start()
        tails.append(c)
    for i, (_, _, ov, vt) in enumerate(vc):
        bulk[4 + i].wait()
        c = pltpu.make_async_copy(vt, ov.at[:, :, :, pl.ds(aligned, 128)], sems[4 + i])
        c.start()
        tails.append(c)
    for c in tails:
        c.wait()


def kernel(input_pos, kv_cache_k_0, kv_slice_k_0, kv_cache_v_0, kv_slice_v_0, kv_cache_k_1, kv_slice_k_1, kv_cache_v_1, kv_slice_v_1, kv_cache_k_2, kv_slice_k_2, kv_cache_v_2, kv_slice_v_2, kv_cache_k_3, kv_slice_k_3, kv_cache_v_3, kv_slice_v_3):
    caches = (kv_cache_k_0, kv_cache_v_0, kv_cache_k_1, kv_cache_v_1,
              kv_cache_k_2, kv_cache_v_2, kv_cache_k_3, kv_cache_v_3)
    slices = (kv_slice_k_0, kv_slice_v_0, kv_slice_k_1, kv_slice_v_1,
              kv_slice_k_2, kv_slice_v_2, kv_slice_k_3, kv_slice_v_3)

    hbm_spec = pl.BlockSpec(memory_space=pltpu.HBM)
    vmem_spec = pl.BlockSpec(memory_space=pltpu.VMEM)
    smem_spec = pl.BlockSpec(memory_space=pltpu.SMEM)

    out = pl.pallas_call(
        _update_body,
        out_shape=tuple(jax.ShapeDtypeStruct(c.shape, c.dtype) for c in caches),
        in_specs=[smem_spec] + [hbm_spec] * 8 + [vmem_spec] * 8,
        out_specs=(hbm_spec,) * 8,
        scratch_shapes=[pltpu.VMEM((1, 4, 256, 128), jnp.float32)] * 4
                       + [pltpu.SemaphoreType.DMA] * 12,
        name="kv_cache_update",
    )(input_pos, *caches, *slices)

    ok0, ov0, ok1, ov1, ok2, ov2, ok3, ov3 = out
    return (ok0, ov0, ok1, ov1, ok2, ov2, ok3, ov3)


# speedup vs baseline: 42.4464x; 42.4464x over previous
import jax
import jax.numpy as jnp
from jax.experimental import pallas as pl
from jax.experimental.pallas import tpu as pltpu

_N = 32
_CK = 4096 // _N


def _stream_body(pos_ref,
                 ck0, cv0, ck1, cv1, ck2, cv2, ck3, cv3,
                 ks0, vs0, ks1, vs1, ks2, vs2, ks3, vs3,
                 ok0, ov0, ok1, ov1, ok2, ov2, ok3, ov3):
    i = pl.program_id(0)
    p = pos_ref[0]
    row = jax.lax.broadcasted_iota(jnp.int32, ck0.shape, 2) + i * _CK
    kmask = row == p
    for ck, ks, ok in ((ck0, ks0, ok0), (ck1, ks1, ok1),
                       (ck2, ks2, ok2), (ck3, ks3, ok3)):
        ok[...] = jnp.where(kmask, ks[...], ck[...])
    lanecol = jax.lax.broadcasted_iota(jnp.int32, cv0.shape, 3) + i * _CK
    vmask = lanecol == p
    for cv, vs, ov in ((cv0, vs0, ov0), (cv1, vs1, ov1),
                       (cv2, vs2, ov2), (cv3, vs3, ov3)):
        ov[...] = jnp.where(vmask, vs[...], cv[...])


def kernel(input_pos, kv_cache_k_0, kv_slice_k_0, kv_cache_v_0, kv_slice_v_0, kv_cache_k_1, kv_slice_k_1, kv_cache_v_1, kv_slice_v_1, kv_cache_k_2, kv_slice_k_2, kv_cache_v_2, kv_slice_v_2, kv_cache_k_3, kv_slice_k_3, kv_cache_v_3, kv_slice_v_3):
    k_caches = (kv_cache_k_0, kv_cache_k_1, kv_cache_k_2, kv_cache_k_3)
    v_caches = (kv_cache_v_0, kv_cache_v_1, kv_cache_v_2, kv_cache_v_3)
    k_slices = (kv_slice_k_0, kv_slice_k_1, kv_slice_k_2, kv_slice_k_3)
    v_slices = (kv_slice_v_0, kv_slice_v_1, kv_slice_v_2, kv_slice_v_3)

    B, H, KV, D = k_caches[0].shape

    kspec = pl.BlockSpec((B, H, _CK, D), lambda i: (0, 0, i, 0))
    vspec = pl.BlockSpec((B, H, D, _CK), lambda i: (0, 0, 0, i))
    kslice_spec = pl.BlockSpec((B, H, 1, D), lambda i: (0, 0, 0, 0))
    vslice_spec = pl.BlockSpec((B, H, D, 1), lambda i: (0, 0, 0, 0))
    smem_spec = pl.BlockSpec(memory_space=pltpu.SMEM)

    caches = (k_caches[0], v_caches[0], k_caches[1], v_caches[1],
              k_caches[2], v_caches[2], k_caches[3], v_caches[3])
    slices = (k_slices[0], v_slices[0], k_slices[1], v_slices[1],
              k_slices[2], v_slices[2], k_slices[3], v_slices[3])

    out = pl.pallas_call(
        _stream_body,
        grid=(_N,),
        out_shape=tuple(jax.ShapeDtypeStruct(c.shape, c.dtype) for c in caches),
        in_specs=[smem_spec] + [kspec, vspec] * 4 + [kslice_spec, vslice_spec] * 4,
        out_specs=(kspec, vspec) * 4,
        name="kv_cache_stream_update",
    )(input_pos, *caches, *slices)

    ok0, ov0, ok1, ov1, ok2, ov2, ok3, ov3 = out
    return (ok0, ov0, ok1, ov1, ok2, ov2, ok3, ov3)


# speedup vs baseline: 43.4069x; 1.0226x over previous
import jax
import jax.numpy as jnp
from jax.experimental import pallas as pl
from jax.experimental.pallas import tpu as pltpu


def _scatter_body(pos_ref,
                  c0, c1, c2, c3, c4, c5, c6, c7,
                  ks0, vs0, ks1, vs1, ks2, vs2, ks3, vs3,
                  ok0, ov0, ok1, ov1, ok2, ov2, ok3, ov3,
                  vt0, vt1, vt2, vt3,
                  *sems):
    del c0, c1, c2, c3, c4, c5, c6, c7
    p = pos_ref[0]
    aligned = pl.multiple_of((p // 128) * 128, 128)
    col = p - aligned
    in_copies = []
    for i, (ov, vt) in enumerate(((ov0, vt0), (ov1, vt1), (ov2, vt2), (ov3, vt3))):
        c = pltpu.make_async_copy(ov.at[:, :, :, pl.ds(aligned, 128)], vt, sems[4 + i])
        c.start()
        in_copies.append(c)
    k_copies = []
    for i, (ks, ok) in enumerate(((ks0, ok0), (ks1, ok1), (ks2, ok2), (ks3, ok3))):
        c = pltpu.make_async_copy(ks, ok.at[:, :, pl.ds(p, 1), :], sems[i])
        c.start()
        k_copies.append(c)
    lane = jax.lax.broadcasted_iota(jnp.int32, vt0.shape, 3)
    out_copies = []
    for i, (vs, ov, vt) in enumerate(((vs0, ov0, vt0), (vs1, ov1, vt1),
                                      (vs2, ov2, vt2), (vs3, ov3, vt3))):
        in_copies[i].wait()
        vt[...] = jnp.where(lane == col, vs[...], vt[...])
        c = pltpu.make_async_copy(vt, ov.at[:, :, :, pl.ds(aligned, 128)], sems[4 + i])
        c.start()
        out_copies.append(c)
    for c in k_copies + out_copies:
        c.wait()


def kernel(input_pos, kv_cache_k_0, kv_slice_k_0, kv_cache_v_0, kv_slice_v_0, kv_cache_k_1, kv_slice_k_1, kv_cache_v_1, kv_slice_v_1, kv_cache_k_2, kv_slice_k_2, kv_cache_v_2, kv_slice_v_2, kv_cache_k_3, kv_slice_k_3, kv_cache_v_3, kv_slice_v_3):
    caches = (kv_cache_k_0, kv_cache_v_0, kv_cache_k_1, kv_cache_v_1,
              kv_cache_k_2, kv_cache_v_2, kv_cache_k_3, kv_cache_v_3)
    slices = (kv_slice_k_0, kv_slice_v_0, kv_slice_k_1, kv_slice_v_1,
              kv_slice_k_2, kv_slice_v_2, kv_slice_k_3, kv_slice_v_3)

    hbm_spec = pl.BlockSpec(memory_space=pltpu.HBM)
    vmem_spec = pl.BlockSpec(memory_space=pltpu.VMEM)
    smem_spec = pl.BlockSpec(memory_space=pltpu.SMEM)
    B, H, D = 1, 4, 256

    out = pl.pallas_call(
        _scatter_body,
        out_shape=tuple(jax.ShapeDtypeStruct(c.shape, c.dtype) for c in caches),
        in_specs=[smem_spec] + [hbm_spec] * 8 + [vmem_spec] * 8,
        out_specs=(hbm_spec,) * 8,
        scratch_shapes=[pltpu.VMEM((B, H, D, 128), jnp.float32)] * 4
                       + [pltpu.SemaphoreType.DMA] * 8,
        input_output_aliases={1 + i: i for i in range(8)},
        name="kv_cache_scatter_update",
    )(input_pos, *caches, *slices)

    ok0, ov0, ok1, ov1, ok2, ov2, ok3, ov3 = out
    return (ok0, ov0, ok1, ov1, ok2, ov2, ok3, ov3)
